# Initial kernel scaffold; baseline (speedup 1.0000x reference)
import jax
import jax.numpy as jnp
from jax.experimental import pallas as pl


def kernel(byte_input, embed, W1, b1, W2, b2):
    raise NotImplementedError("write your pallas kernel here")


# speedup vs baseline: 1.6446x; 1.6446x over previous
import functools

import jax
import jax.numpy as jnp
from jax import lax
from jax.experimental import pallas as pl
from jax.experimental.pallas import tpu as pltpu
from jax.experimental.pallas import tpu_sc as plsc

_D_MODEL = 384
_VOCAB = 300
_VPAD = 384

_NC = 2
_NS = 16
_NW = _NC * _NS
_CHUNK = 128


def _table_body(embed_ref, w1_ref, b1_ref, w2_ref, b2_ref, out_ref):
    h = lax.dot(embed_ref[...], w1_ref[...],
                precision=lax.Precision.HIGHEST,
                preferred_element_type=jnp.float32) + b1_ref[...]
    out_ref[...] = lax.dot(h, w2_ref[...],
                           precision=lax.Precision.HIGHEST,
                           preferred_element_type=jnp.float32) + b2_ref[...]


def _make_table(embed, W1, b1, W2, b2):
    W2p = jnp.pad(W2, ((0, 0), (0, _VPAD - _VOCAB)))
    b2p = jnp.pad(b2, (0, _VPAD - _VOCAB))
    return pl.pallas_call(
        _table_body,
        out_shape=jax.ShapeDtypeStruct((_VOCAB, _VPAD), jnp.float32),
    )(embed, W1, b1.reshape(1, _D_MODEL), W2p, b2p.reshape(1, _VPAD))


def _make_gather(n_tokens):
    per_w = n_tokens // _NW
    n_chunks = per_w // _CHUNK
    mesh = plsc.VectorSubcoreMesh(core_axis_name="c", subcore_axis_name="s")

    @functools.partial(
        pl.kernel, mesh=mesh,
        out_type=jax.ShapeDtypeStruct((n_tokens, _VPAD), jnp.float32),
        scratch_types=[
            pltpu.VMEM((_CHUNK,), jnp.int32),
            pltpu.VMEM((_CHUNK, _VPAD), jnp.float32),
            pltpu.SemaphoreType.DMA,
        ],
    )
    def gather(table_hbm, idx_hbm, out_hbm, idx_v, rows_v, sem):
        wid = lax.axis_index("s") * _NC + lax.axis_index("c")
        base = wid * per_w
        for c in range(n_chunks):
            off = base + c * _CHUNK
            pltpu.sync_copy(idx_hbm.at[pl.ds(off, _CHUNK)], idx_v)
            pltpu.async_copy(table_hbm.at[idx_v], rows_v, sem).wait()
            pltpu.sync_copy(rows_v, out_hbm.at[pl.ds(off, _CHUNK)])

    return gather


def kernel(byte_input, embed, W1, b1, W2, b2):
    batch, seq = byte_input.shape
    table = _make_table(embed, W1, b1, W2, b2)
    ids = byte_input.reshape(-1).astype(jnp.int32)
    out = _make_gather(batch * seq)(table, ids)
    return out[:, :_VOCAB].reshape(batch, seq, _VOCAB)
